# baseline (speedup 1.0000x reference)
import jax
import jax.numpy as jnp
from jax.experimental import pallas as pl

_N = 1000000
_V = 32
_B = 131072


def _vote_block(x_ref, o_ref):
    s = jnp.sum(x_ref[...], axis=0)
    o_ref[...] = (s > _V // 2).astype(jnp.int32)


def kernel(inputs):
    n_blocks = (_N + _B - 1) // _B
    out = pl.pallas_call(
        _vote_block,
        grid=(n_blocks,),
        in_specs=[pl.BlockSpec((_V, _B), lambda i: (0, i))],
        out_specs=pl.BlockSpec((_B,), lambda i: (i,)),
        out_shape=jax.ShapeDtypeStruct((_N,), jnp.int32),
    )(inputs)
    return out


# speedup vs baseline: 1.0412x; 1.0412x over previous
import jax
import jax.numpy as jnp
from jax.experimental import pallas as pl

_N = 1000000
_V = 32
_B = 71680


def _vote_block(x_ref, o_ref):
    s = jnp.sum(x_ref[...], axis=0)
    o_ref[...] = (s > _V // 2).astype(jnp.int32)


def kernel(inputs):
    n_blocks = (_N + _B - 1) // _B
    out = pl.pallas_call(
        _vote_block,
        grid=(n_blocks,),
        in_specs=[pl.BlockSpec((_V, _B), lambda i: (0, i))],
        out_specs=pl.BlockSpec((_B,), lambda i: (i,)),
        out_shape=jax.ShapeDtypeStruct((_N,), jnp.int32),
    )(inputs)
    return out


# speedup vs baseline: 1.0479x; 1.0064x over previous
import jax
import jax.numpy as jnp
from jax.experimental import pallas as pl

_N = 1000000
_V = 32
_B = 65536


def _vote_block(x_ref, o_ref):
    s = jnp.sum(x_ref[...], axis=0)
    o_ref[...] = (s > _V // 2).astype(jnp.int32)


def kernel(inputs):
    n_blocks = (_N + _B - 1) // _B
    return pl.pallas_call(
        _vote_block,
        grid=(n_blocks,),
        in_specs=[pl.BlockSpec((_V, _B), lambda i: (0, i))],
        out_specs=pl.BlockSpec((_B,), lambda i: (i,)),
        out_shape=jax.ShapeDtypeStruct((_N,), jnp.int32),
    )(inputs)
